# Initial kernel scaffold; baseline (speedup 1.0000x reference)
import jax
import jax.numpy as jnp
from jax.experimental import pallas as pl


def kernel(slots_attn_mask, slots):
    raise NotImplementedError("write your pallas kernel here")


# speedup vs baseline: 2.1410x; 2.1410x over previous
import math

import jax
import jax.numpy as jnp
from jax import lax
from jax.experimental import pallas as pl

_K = 5
_EPS = 1e-08


def _fused_kernel(mask_ref, slots_ref, out_ref):
    b = pl.program_id(0)
    nb = pl.num_programs(0)
    x = mask_ref[0]
    T, S, N = x.shape
    logN = math.log(N)

    xl = x * jnp.log(x + _EPS)
    ent = jnp.sum(xl, axis=2).sum(axis=0, keepdims=True)
    psum = jnp.sum(x, axis=2).sum(axis=0, keepdims=True)
    klmean = ent / T + logN
    klsum = ent + logN * psum

    y = slots_ref[0]
    norm = jnp.sqrt(jnp.sum(y * y, axis=1, keepdims=True))
    yn = y / jnp.maximum(norm, 1e-12)
    sim = lax.dot_general(yn, yn, (((1,), (1,)), ((), ())),
                          preferred_element_type=jnp.float32)

    irow = lax.broadcasted_iota(jnp.int32, (S, S), 0)
    jcol = lax.broadcasted_iota(jnp.int32, (S, S), 1)
    tri = jcol > irow
    flat = irow * S + jcol
    li = lax.broadcasted_iota(jnp.int32, (1, S), 1)

    def lookup(vec, idx):
        return jnp.sum(jnp.where(li == idx, vec, 0.0))

    banned = jnp.zeros((S, S), dtype=jnp.bool_)
    total = 0.0
    for _ in range(_K):
        valid = tri & jnp.logical_not(banned)
        masked = jnp.where(valid, sim, -jnp.inf)
        mval = jnp.max(masked)
        cand = jnp.where(masked == mval, flat, S * S)
        fidx = jnp.min(cand)
        ii = fidx // S
        jj = fidx - ii * S
        k_i = lookup(klmean, ii)
        k_j = lookup(klmean, jj)
        chosen = jnp.where(k_i <= k_j, ii, jj)
        total = total + lookup(klsum, chosen)
        banned = banned | (irow == chosen) | (jcol == chosen)

    scale = 1.0 / (nb * T * _K)

    @pl.when(b == 0)
    def _():
        out_ref[...] = jnp.zeros_like(out_ref)

    out_ref[...] += jnp.reshape(total * scale, (1, 1))


def kernel(slots_attn_mask, slots):
    B, T, S, N = slots_attn_mask.shape
    D = slots.shape[-1]
    slots_last = slots[:, -1]
    loss = pl.pallas_call(
        _fused_kernel,
        grid=(B,),
        in_specs=[
            pl.BlockSpec((1, T, S, N), lambda b: (b, 0, 0, 0)),
            pl.BlockSpec((1, S, D), lambda b: (b, 0, 0)),
        ],
        out_specs=pl.BlockSpec((1, 1), lambda b: (0, 0)),
        out_shape=jax.ShapeDtypeStruct((1, 1), jnp.float32),
    )(slots_attn_mask, slots_last)
    return loss[0, 0]
